# Initial kernel scaffold; baseline (speedup 1.0000x reference)
import jax
import jax.numpy as jnp
from jax.experimental import pallas as pl


def kernel(x, gate_idx, W, b):
    raise NotImplementedError("write your pallas kernel here")


# speedup vs baseline: 1.4474x; 1.4474x over previous
import functools

import jax
import jax.numpy as jnp
from jax import lax
from jax.experimental import pallas as pl
from jax.experimental.pallas import tpu as pltpu
from jax.experimental.pallas import tpu_sc as plsc

BLK = 256

SC_CORES = 2
SC_SUBCORES = 16
SC_WORKERS = SC_CORES * SC_SUBCORES


def _sc_row_gather(table, idx):
    n_rows, d = table.shape
    b = idx.shape[0]
    rows_per_w = b // SC_WORKERS
    mesh = plsc.VectorSubcoreMesh(core_axis_name="c", subcore_axis_name="s")

    @functools.partial(
        pl.kernel,
        mesh=mesh,
        out_type=jax.ShapeDtypeStruct((b, d), table.dtype),
        scratch_types=[
            pltpu.VMEM((rows_per_w,), jnp.int32),
            pltpu.VMEM((rows_per_w, d), table.dtype),
            pltpu.SemaphoreType.DMA,
        ],
    )
    def gather_kernel(table_hbm, idx_hbm, out_hbm, idx_v, rows_v, sem):
        wid = lax.axis_index("s") * SC_CORES + lax.axis_index("c")
        base = wid * rows_per_w
        pltpu.sync_copy(idx_hbm.at[pl.ds(base, rows_per_w)], idx_v)
        pltpu.async_copy(table_hbm.at[idx_v], rows_v, sem).wait()
        pltpu.sync_copy(rows_v, out_hbm.at[pl.ds(base, rows_per_w)])

    return gather_kernel(table, idx)


def _mm_body(tile_ids, expert_sel, expert_cmp, xs_ref, w_ref, b_ref, es_ref,
             out_ref):
    i = pl.program_id(0)
    t_cur = tile_ids[i]
    t_prev = tile_ids[jnp.maximum(i - 1, 0)]
    first_visit = jnp.logical_or(i == 0, t_cur != t_prev)
    ecmp = expert_cmp[i]

    @pl.when(first_visit)
    def _():
        out_ref[...] = jnp.zeros_like(out_ref)

    @pl.when(ecmp >= 0)
    def _():
        mask = es_ref[0] == ecmp
        acc = jnp.dot(xs_ref[...], w_ref[0],
                      preferred_element_type=jnp.float32)
        contrib = jnp.where(mask, acc + b_ref[0], 0.0)
        out_ref[...] += contrib


def _grouped_matmul(xs, w, b3, es3, tile_ids, expert_sel, expert_cmp):
    n, d = xs.shape
    num_items = tile_ids.shape[0]
    grid_spec = pltpu.PrefetchScalarGridSpec(
        num_scalar_prefetch=3,
        grid=(num_items,),
        in_specs=[
            pl.BlockSpec((BLK, d), lambda i, t, es, ec: (t[i], 0)),
            pl.BlockSpec((1, d, d), lambda i, t, es, ec: (es[i], 0, 0)),
            pl.BlockSpec((1, 1, d), lambda i, t, es, ec: (es[i], 0, 0)),
            pl.BlockSpec((1, BLK, 1), lambda i, t, es, ec: (t[i], 0, 0)),
        ],
        out_specs=pl.BlockSpec((BLK, d), lambda i, t, es, ec: (t[i], 0)),
    )
    return pl.pallas_call(
        _mm_body,
        grid_spec=grid_spec,
        out_shape=jax.ShapeDtypeStruct((n, d), jnp.float32),
        compiler_params=pltpu.CompilerParams(
            dimension_semantics=("arbitrary",)),
    )(tile_ids, expert_sel, expert_cmp, xs, w, b3, es3)


def kernel(x, gate_idx, W, b):
    n, d = x.shape
    e_total = W.shape[0]
    num_tiles = n // BLK
    num_items = num_tiles + e_total - 1

    e = gate_idx[:, 0].astype(jnp.int32)
    iota = jnp.arange(n, dtype=jnp.int32)
    e_sorted, perm = lax.sort_key_val(e, iota)
    inv_perm = jnp.zeros((n,), jnp.int32).at[perm].set(iota)

    counts = jnp.sum(
        (e[None, :] == jnp.arange(e_total, dtype=jnp.int32)[:, None]).astype(
            jnp.int32),
        axis=1)
    ends = jnp.cumsum(counts).astype(jnp.int32)
    starts = ends - counts

    pair = jnp.arange(num_tiles * e_total, dtype=jnp.int32)
    t_ids = pair // e_total
    e_ids = pair % e_total
    seg_start = starts[e_ids]
    seg_end = ends[e_ids]
    blk_start = t_ids * BLK
    valid = ((seg_start < blk_start + BLK) & (seg_end > blk_start)
             & (seg_end > seg_start))
    key = jnp.where(valid, pair, jnp.int32(num_tiles * e_total))
    key = jnp.sort(key)[:num_items]
    is_real = key < num_tiles * e_total
    tile_ids = jnp.where(is_real, key // e_total, num_tiles - 1)
    expert_cmp = jnp.where(is_real, key % e_total, -1)
    expert_sel = jnp.where(is_real, key % e_total, e_total - 1)

    xs = _sc_row_gather(x, perm)
    es3 = e_sorted.reshape(num_tiles, BLK, 1)
    b3 = b.reshape(e_total, 1, d)
    ys = _grouped_matmul(xs, W, b3, es3, tile_ids, expert_sel, expert_cmp)
    return _sc_row_gather(ys, inv_perm)
